# baseline (speedup 1.0000x reference)
import functools

import jax
import jax.numpy as jnp
from jax import lax
from jax.experimental import pallas as pl
from jax.experimental.pallas import tpu as pltpu
from jax.experimental.pallas import tpu_sc as plsc

EPS = 0.001
NSUP = 601
SMIN = -300.0

B, K = 4096, 50
GB = 4
GVALS = GB * K
NCH = (GVALS + 15) // 16
TPAD = NCH * 16
NW = 32
GROUPS = B // GB
GPW = GROUPS // NW
BPW = B // NW
SLAB = GB * K * NSUP


def _tt_kernel(tv_ref, tt_ref):
    x = tv_ref[...]
    tt_ref[...] = jnp.sign(x) * (jnp.sqrt(jnp.abs(x) + 1.0) - 1.0 + EPS * x)


def _sc_expand(tt_hbm, zeros_hbm, qoff_hbm, out_hbm, rowbuf, ttv, qoffv):
    c = lax.axis_index("c")
    s = lax.axis_index("s")
    wid = s * 2 + c

    pltpu.sync_copy(qoff_hbm, qoffv)
    pltpu.sync_copy(tt_hbm.at[pl.ds(wid * BPW * K, BPW * K)], ttv)
    pltpu.sync_copy(zeros_hbm, rowbuf)

    lane = lax.iota(jnp.int32, 16)

    def group_body(j, carry):
        g = wid * GPW + j

        idx_lo = []
        idx_hi = []
        for ch in range(NCH):
            o = ch * 16
            t16 = ttv[pl.ds(j * GVALS + o, 16)]
            q16 = qoffv[pl.ds(o, 16)]
            mask = (lane + o) < GVALS
            pos = t16 - SMIN
            li = pos.astype(jnp.int32)
            li = jnp.minimum(jnp.maximum(li, 0), NSUP - 1)
            ui = jnp.minimum(li + 1, NSUP - 1)
            lowf = li.astype(jnp.float32)
            p_low = lowf + 1.0 - pos
            p_high = pos - lowf
            il = q16 + li
            iu = q16 + ui
            plsc.store_scatter(rowbuf, [il], p_low, mask=mask)
            plsc.store_scatter(rowbuf, [iu], p_high, mask=mask)
            idx_lo.append((il, mask))
            idx_hi.append((iu, mask))

        pltpu.sync_copy(rowbuf, out_hbm.at[pl.ds(g * SLAB, SLAB)])

        z16 = jnp.zeros((16,), jnp.float32)
        for (il, mask), (iu, _) in zip(idx_lo, idx_hi):
            plsc.store_scatter(rowbuf, [il], z16, mask=mask)
            plsc.store_scatter(rowbuf, [iu], z16, mask=mask)
        return carry

    lax.fori_loop(0, GPW, group_body, 0)


def kernel(target_value, supports):
    tt = pl.pallas_call(
        _tt_kernel,
        out_shape=jax.ShapeDtypeStruct((B * K // 128, 128), jnp.float32),
    )(target_value.reshape(B * K // 128, 128))
    tt_flat = tt.reshape(B * K)

    q = jnp.arange(TPAD, dtype=jnp.int32) % GVALS
    qoff = q * NSUP
    zeros = jnp.zeros((SLAB,), jnp.float32)

    mesh = plsc.VectorSubcoreMesh(core_axis_name="c", subcore_axis_name="s")
    sck = functools.partial(
        pl.kernel,
        mesh=mesh,
        out_type=jax.ShapeDtypeStruct((B * K * NSUP,), jnp.float32),
        compiler_params=pltpu.CompilerParams(
            use_tc_tiling_on_sc=False, needs_layout_passes=False),
        scratch_types=[
            pltpu.VMEM((SLAB,), jnp.float32),
            pltpu.VMEM((BPW * K,), jnp.float32),
            pltpu.VMEM((TPAD,), jnp.int32),
        ],
    )(_sc_expand)
    return sck(tt_flat, zeros, qoff).reshape(B, K, NSUP)


# speedup vs baseline: 3.3299x; 3.3299x over previous
import functools

import jax
import jax.numpy as jnp
from jax import lax
from jax.experimental import pallas as pl
from jax.experimental.pallas import tpu as pltpu
from jax.experimental.pallas import tpu_sc as plsc

EPS = 0.001
NSUP = 601
SMIN = -300.0

B, K = 4096, 50
GB = 2
GVALS = GB * K
NCH = (GVALS + 15) // 16
TPAD = NCH * 16
NW = 32
GROUPS = B // GB
GPW = GROUPS // NW
BPW = B // NW


def _tt_kernel(tv_ref, tt_ref):
    x = tv_ref[...]
    tt_ref[...] = jnp.sign(x) * (jnp.sqrt(jnp.abs(x) + 1.0) - 1.0 + EPS * x)


def _sc_expand(tt_hbm, zeros_hbm, bq_hbm, kq_hbm, out_hbm,
               rowbuf, ttv, bqv, kqv):
    c = lax.axis_index("c")
    s = lax.axis_index("s")
    wid = s * 2 + c

    pltpu.sync_copy(bq_hbm, bqv)
    pltpu.sync_copy(kq_hbm, kqv)
    pltpu.sync_copy(tt_hbm.at[pl.ds(wid * BPW * K, BPW * K)], ttv)
    pltpu.sync_copy(zeros_hbm, rowbuf)

    lane = lax.iota(jnp.int32, 16)

    def group_body(j, carry):
        b0 = wid * BPW + j * GB

        poked = []
        for ch in range(NCH):
            o = ch * 16
            t16 = ttv[pl.ds(j * GVALS + o, 16)]
            b16 = bqv[pl.ds(o, 16)]
            k16 = kqv[pl.ds(o, 16)]
            mask = (lane + o) < GVALS
            pos = t16 - SMIN
            li = pos.astype(jnp.int32)
            li = jnp.minimum(jnp.maximum(li, 0), NSUP - 1)
            ui = jnp.minimum(li + 1, NSUP - 1)
            lowf = li.astype(jnp.float32)
            p_low = lowf + 1.0 - pos
            p_high = pos - lowf
            plsc.store_scatter(rowbuf, [b16, k16, li], p_low, mask=mask)
            plsc.store_scatter(rowbuf, [b16, k16, ui], p_high, mask=mask)
            poked.append((b16, k16, li, ui, mask))

        pltpu.sync_copy(rowbuf, out_hbm.at[pl.ds(b0, GB)])

        z16 = jnp.zeros((16,), jnp.float32)
        for b16, k16, li, ui, mask in poked:
            plsc.store_scatter(rowbuf, [b16, k16, li], z16, mask=mask)
            plsc.store_scatter(rowbuf, [b16, k16, ui], z16, mask=mask)
        return carry

    lax.fori_loop(0, GPW, group_body, 0)


def kernel(target_value, supports):
    tt = pl.pallas_call(
        _tt_kernel,
        out_shape=jax.ShapeDtypeStruct((B * K // 128, 128), jnp.float32),
    )(target_value.reshape(B * K // 128, 128))
    tt_flat = tt.reshape(B * K)

    q = jnp.arange(TPAD, dtype=jnp.int32) % GVALS
    bq = q // K
    kq = q % K
    zeros = jnp.zeros((GB, K, NSUP), jnp.float32)

    mesh = plsc.VectorSubcoreMesh(core_axis_name="c", subcore_axis_name="s")
    sck = functools.partial(
        pl.kernel,
        mesh=mesh,
        out_type=jax.ShapeDtypeStruct((B, K, NSUP), jnp.float32),
        compiler_params=pltpu.CompilerParams(needs_layout_passes=False),
        scratch_types=[
            pltpu.VMEM((GB, K, NSUP), jnp.float32),
            pltpu.VMEM((BPW * K,), jnp.float32),
            pltpu.VMEM((TPAD,), jnp.int32),
            pltpu.VMEM((TPAD,), jnp.int32),
        ],
    )(_sc_expand)
    return sck(tt_flat, zeros, bq, kq)


# speedup vs baseline: 13.7285x; 4.1228x over previous
import jax
import jax.numpy as jnp
from jax import lax
from jax.experimental import pallas as pl

EPS = 0.001
NS = 601
SMIN = -300.0

B, K = 4096, 50
CB = 8


def _tt_kernel(tv_ref, tt_ref):
    x = tv_ref[...]
    tt_ref[...] = jnp.sign(x) * (jnp.sqrt(jnp.abs(x) + 1.0) - 1.0 + EPS * x)


def _tent_block(tt_ref, out_ref, *, cb):
    j = pl.program_id(0)
    tt = tt_ref[...]
    col = lax.broadcasted_iota(jnp.int32, (K, cb, B), 1) + j * cb
    sup = col.astype(jnp.float32) + SMIN
    out_ref[...] = jnp.maximum(1.0 - jnp.abs(sup - tt[:, None, :]), 0.0)


def kernel(target_value, supports):
    tt = pl.pallas_call(
        _tt_kernel,
        out_shape=jax.ShapeDtypeStruct((B * K // 128, 128), jnp.float32),
    )(target_value.reshape(B * K // 128, 128))
    tt_t = tt.reshape(B, K).T

    import functools
    grid = (NS + CB - 1) // CB
    out = pl.pallas_call(
        functools.partial(_tent_block, cb=CB),
        grid=(grid,),
        in_specs=[pl.BlockSpec((K, B), lambda j: (0, 0))],
        out_specs=pl.BlockSpec((K, CB, B), lambda j: (0, j, 0)),
        out_shape=jax.ShapeDtypeStruct((K, NS, B), jnp.float32),
    )(tt_t)
    return jnp.transpose(out, (2, 0, 1))


# speedup vs baseline: 14.5072x; 1.0567x over previous
import jax
import jax.numpy as jnp
from jax import lax
from jax.experimental import pallas as pl

EPS = 0.001
NS = 601
SMIN = -300.0

B, K = 4096, 50
CB = 16


def _tt_kernel(tv_ref, tt_ref):
    x = tv_ref[...]
    tt_ref[...] = jnp.sign(x) * (jnp.sqrt(jnp.abs(x) + 1.0) - 1.0 + EPS * x)


def _tent_block(tt_ref, out_ref, *, cb):
    j = pl.program_id(0)
    tt = tt_ref[...]
    col = lax.broadcasted_iota(jnp.int32, (K, cb, B), 1) + j * cb
    sup = col.astype(jnp.float32) + SMIN
    out_ref[...] = jnp.maximum(1.0 - jnp.abs(sup - tt[:, None, :]), 0.0)


def kernel(target_value, supports):
    tt = pl.pallas_call(
        _tt_kernel,
        out_shape=jax.ShapeDtypeStruct((B * K // 128, 128), jnp.float32),
    )(target_value.reshape(B * K // 128, 128))
    tt_t = tt.reshape(B, K).T

    import functools
    grid = (NS + CB - 1) // CB
    out = pl.pallas_call(
        functools.partial(_tent_block, cb=CB),
        grid=(grid,),
        in_specs=[pl.BlockSpec((K, B), lambda j: (0, 0))],
        out_specs=pl.BlockSpec((K, CB, B), lambda j: (0, j, 0)),
        out_shape=jax.ShapeDtypeStruct((K, NS, B), jnp.float32),
    )(tt_t)
    return jnp.transpose(out, (2, 0, 1))
